# baseline (speedup 1.0000x reference)
import functools

import jax
import jax.numpy as jnp
from jax import lax
from jax.experimental import pallas as pl
from jax.experimental.pallas import tpu as pltpu
from jax.experimental.pallas import tpu_sc as plsc

NATOMS = 6400000
NSTRUCT = 1024
NSPEC = 119
NSPEC_PAD = 128

NC = 2
NS = 16
L = 16
NW = NC * NS

ATOMS_PER_W = NATOMS // NW
CHUNK = 10000
NCHUNK = ATOMS_PER_W // CHUNK
VREGS = CHUNK // L
ROWSTRIDE = NSTRUCT + L

_mesh = plsc.VectorSubcoreMesh(
    core_axis_name="c", subcore_axis_name="s", num_cores=NC, num_subcores=NS
)
_params = pltpu.CompilerParams(needs_layout_passes=False)


@functools.partial(
    pl.kernel,
    out_type=jax.ShapeDtypeStruct((NW * NSTRUCT,), jnp.float32),
    mesh=_mesh,
    compiler_params=_params,
    scratch_types=[
        pltpu.VMEM((NSPEC_PAD * L,), jnp.int32),
        pltpu.VMEM((NSPEC_PAD,), jnp.float32),
        pltpu.VMEM((NSPEC_PAD,), jnp.float32),
        pltpu.VMEM((CHUNK,), jnp.float32),
        pltpu.VMEM((CHUNK,), jnp.int32),
        pltpu.VMEM((CHUNK,), jnp.int32),
        pltpu.VMEM((CHUNK,), jnp.float32),
        pltpu.VMEM((CHUNK,), jnp.int32),
        pltpu.VMEM((CHUNK,), jnp.int32),
        pltpu.VMEM((2 * L * ROWSTRIDE,), jnp.float32),
        pltpu.VMEM((NSTRUCT,), jnp.float32),
        pltpu.SemaphoreType.DMA,
        pltpu.SemaphoreType.DMA,
    ],
)
def _partials_kernel(le_hbm, z_hbm, b_hbm, scale_hbm, shift_hbm, out_hbm,
                     tbl_v, scale_sv, shift_sv, le_a, z_a, b_a, le_b, z_b, b_b,
                     acc_v, fold_v, sem_a, sem_b):
    wid = lax.axis_index("s") * NC + lax.axis_index("c")
    lane = lax.iota(jnp.int32, L)

    w_base = wid * ATOMS_PER_W

    def start(ci, le_v, z_v, b_v, sem):
        base = pl.multiple_of(w_base + ci * CHUNK, CHUNK)
        pltpu.async_copy(le_hbm.at[pl.ds(base, CHUNK)], le_v, sem)
        pltpu.async_copy(z_hbm.at[pl.ds(base, CHUNK)], z_v, sem)
        pltpu.async_copy(b_hbm.at[pl.ds(base, CHUNK)], b_v, sem)

    def wait(le_v, z_v, b_v, sem):
        pltpu.make_async_copy(le_hbm.at[pl.ds(0, CHUNK)], le_v, sem).wait()
        pltpu.make_async_copy(z_hbm.at[pl.ds(0, CHUNK)], z_v, sem).wait()
        pltpu.make_async_copy(b_hbm.at[pl.ds(0, CHUNK)], b_v, sem).wait()

    start(0, le_a, z_a, b_a, sem_a)
    pltpu.sync_copy(scale_hbm, scale_sv.at[pl.ds(0, NSPEC)])
    pltpu.sync_copy(shift_hbm, shift_sv.at[pl.ds(0, NSPEC)])

    def rne_hi16(w):
        return (w + 32767 + ((w >> 16) & 1)) & jnp.int32(-65536)

    for j in range(NSPEC_PAD // L):
        ws = plsc.bitcast(scale_sv[pl.ds(j * L, L)], jnp.int32)
        wh = plsc.bitcast(shift_sv[pl.ds(j * L, L)], jnp.int32)
        packed = rne_hi16(ws) | ((rne_hi16(wh) >> 16) & 65535)
        zbase = (j * L + lane) << 4
        for r in range(L):
            plsc.store_scatter(tbl_v, [zbase + ((lane + r) & 15)], packed)

    zero16 = jnp.zeros((L,), jnp.float32)

    @plsc.parallel_loop(0, 2 * L * ROWSTRIDE // L, unroll=8)
    def _(i):
        acc_v[pl.ds(pl.multiple_of(i * L, L), L)] = zero16

    lane_rot = lane * (ROWSTRIDE + 1)

    def compute(le_v, z_v, b_v):
        @plsc.parallel_loop(0, VREGS, unroll=8)
        def _(i):
            sl = pl.ds(pl.multiple_of(i * L, L), L)
            zi = (z_v[sl] << 4) + lane
            bi = b_v[sl] + lane_rot + (i & 1) * (L * ROWSTRIDE)
            pv = plsc.load_gather(tbl_v, [zi])
            s16 = plsc.bitcast(pv & jnp.int32(-65536), jnp.float32)
            sh16 = plsc.bitcast(pv << 16, jnp.float32)
            e16 = le_v[sl] * s16 + sh16
            plsc.addupdate_scatter(acc_v, [bi], e16)

    def pair_body(k, _):
        wait(le_a, z_a, b_a, sem_a)
        start(2 * k + 1, le_b, z_b, b_b, sem_b)
        compute(le_a, z_a, b_a)
        wait(le_b, z_b, b_b, sem_b)
        start(jnp.minimum(2 * k + 2, NCHUNK - 1), le_a, z_a, b_a, sem_a)
        compute(le_b, z_b, b_b)
        return 0

    lax.fori_loop(0, NCHUNK // 2, pair_body, 0)
    wait(le_a, z_a, b_a, sem_a)

    @plsc.parallel_loop(0, NSTRUCT // L, unroll=4)
    def _(k):
        col = k * L
        tot = acc_v[pl.ds(col, L)]
        tot = tot + acc_v[pl.ds(col + L * ROWSTRIDE, L)]
        for r in range(1, L):
            tot = tot + acc_v[pl.ds(col + r * (ROWSTRIDE + 1), L)]
            tot = tot + acc_v[pl.ds(col + r * (ROWSTRIDE + 1) + L * ROWSTRIDE, L)]
        fold_v[pl.ds(pl.multiple_of(col, L), L)] = tot

    pltpu.sync_copy(fold_v, out_hbm.at[pl.ds(wid * NSTRUCT, NSTRUCT)])


def _tc_reduce_body(parts_ref, out_ref):
    @pl.when(pl.program_id(0) == 0)
    def _():
        out_ref[...] = jnp.zeros_like(out_ref)

    out_ref[...] += parts_ref[...]


_tc_reduce = pl.pallas_call(
    _tc_reduce_body,
    grid=(NW,),
    in_specs=[pl.BlockSpec((NSTRUCT,), lambda i: (i,))],
    out_specs=pl.BlockSpec((NSTRUCT,), lambda i: (0,)),
    out_shape=jax.ShapeDtypeStruct((NSTRUCT,), jnp.float32),
)


@jax.jit
def kernel(local_energies, Z, batch, scale, shift):
    z32 = Z.astype(jnp.int32)
    b32 = batch.astype(jnp.int32)
    parts = _partials_kernel(local_energies, z32, b32,
                             scale.astype(jnp.float32),
                             shift.astype(jnp.float32))
    return _tc_reduce(parts)


# speedup vs baseline: 1.1449x; 1.1449x over previous
import functools

import jax
import jax.numpy as jnp
from jax import lax
from jax.experimental import pallas as pl
from jax.experimental.pallas import tpu as pltpu
from jax.experimental.pallas import tpu_sc as plsc

NATOMS = 6400000
NSTRUCT = 1024
NSPEC_PAD = 128

NC = 2
NS = 16
L = 16
NW = NC * NS

ATOMS_PER_W = NATOMS // NW
CHUNK = 10000
NCHUNK = ATOMS_PER_W // CHUNK
VREGS = CHUNK // L
ROWSTRIDE = NSTRUCT + L

_mesh = plsc.VectorSubcoreMesh(
    core_axis_name="c", subcore_axis_name="s", num_cores=NC, num_subcores=NS
)
_params = pltpu.CompilerParams(needs_layout_passes=False)


@functools.partial(
    pl.kernel,
    out_type=jax.ShapeDtypeStruct((NW * NSTRUCT,), jnp.float32),
    mesh=_mesh,
    compiler_params=_params,
    scratch_types=[
        pltpu.VMEM((NSPEC_PAD * L,), jnp.int32),
        pltpu.VMEM((CHUNK,), jnp.float32),
        pltpu.VMEM((CHUNK,), jnp.int32),
        pltpu.VMEM((CHUNK,), jnp.int32),
        pltpu.VMEM((CHUNK,), jnp.float32),
        pltpu.VMEM((CHUNK,), jnp.int32),
        pltpu.VMEM((CHUNK,), jnp.int32),
        pltpu.VMEM((2 * L * ROWSTRIDE,), jnp.float32),
        pltpu.VMEM((NSTRUCT,), jnp.float32),
        pltpu.SemaphoreType.DMA,
        pltpu.SemaphoreType.DMA,
    ],
)
def _partials_kernel(le_hbm, z_hbm, b_hbm, tbl_hbm, out_hbm,
                     tbl_v, le_a, z_a, b_a, le_b, z_b, b_b,
                     acc_v, fold_v, sem_a, sem_b):
    wid = lax.axis_index("s") * NC + lax.axis_index("c")

    pltpu.sync_copy(tbl_hbm, tbl_v)

    zero16 = jnp.zeros((L,), jnp.float32)

    @plsc.parallel_loop(0, 2 * L * ROWSTRIDE // L, unroll=8)
    def _(i):
        acc_v[pl.ds(pl.multiple_of(i * L, L), L)] = zero16

    lane = lax.iota(jnp.int32, L)
    lane_rot = lane * (ROWSTRIDE + 1)
    w_base = wid * ATOMS_PER_W

    def start(ci, le_v, z_v, b_v, sem):
        base = pl.multiple_of(w_base + ci * CHUNK, CHUNK)
        pltpu.async_copy(le_hbm.at[pl.ds(base, CHUNK)], le_v, sem)
        pltpu.async_copy(z_hbm.at[pl.ds(base, CHUNK)], z_v, sem)
        pltpu.async_copy(b_hbm.at[pl.ds(base, CHUNK)], b_v, sem)

    def wait(le_v, z_v, b_v, sem):
        pltpu.make_async_copy(le_hbm.at[pl.ds(0, CHUNK)], le_v, sem).wait()
        pltpu.make_async_copy(z_hbm.at[pl.ds(0, CHUNK)], z_v, sem).wait()
        pltpu.make_async_copy(b_hbm.at[pl.ds(0, CHUNK)], b_v, sem).wait()

    def compute(le_v, z_v, b_v):
        @plsc.parallel_loop(0, VREGS, unroll=8)
        def _(i):
            sl = pl.ds(pl.multiple_of(i * L, L), L)
            zi = (z_v[sl] << 4) + lane
            bi = b_v[sl] + lane_rot + (i & 1) * (L * ROWSTRIDE)
            pv = plsc.load_gather(tbl_v, [zi])
            s16 = plsc.bitcast(pv & jnp.int32(-65536), jnp.float32)
            sh16 = plsc.bitcast(pv << 16, jnp.float32)
            e16 = le_v[sl] * s16 + sh16
            plsc.addupdate_scatter(acc_v, [bi], e16)

    start(0, le_a, z_a, b_a, sem_a)

    def pair_body(k, _):
        wait(le_a, z_a, b_a, sem_a)
        start(2 * k + 1, le_b, z_b, b_b, sem_b)
        compute(le_a, z_a, b_a)
        wait(le_b, z_b, b_b, sem_b)
        start(jnp.minimum(2 * k + 2, NCHUNK - 1), le_a, z_a, b_a, sem_a)
        compute(le_b, z_b, b_b)
        return 0

    lax.fori_loop(0, NCHUNK // 2, pair_body, 0)
    wait(le_a, z_a, b_a, sem_a)

    @plsc.parallel_loop(0, NSTRUCT // L, unroll=4)
    def _(k):
        col = k * L
        tot = acc_v[pl.ds(col, L)]
        tot = tot + acc_v[pl.ds(col + L * ROWSTRIDE, L)]
        for r in range(1, L):
            tot = tot + acc_v[pl.ds(col + r * (ROWSTRIDE + 1), L)]
            tot = tot + acc_v[pl.ds(col + r * (ROWSTRIDE + 1) + L * ROWSTRIDE, L)]
        fold_v[pl.ds(pl.multiple_of(col, L), L)] = tot

    pltpu.sync_copy(fold_v, out_hbm.at[pl.ds(wid * NSTRUCT, NSTRUCT)])


@functools.partial(
    pl.kernel,
    out_type=jax.ShapeDtypeStruct((NSTRUCT,), jnp.float32),
    mesh=_mesh,
    compiler_params=_params,
    scratch_types=[
        pltpu.VMEM((2 * NW, L), jnp.float32),
        pltpu.VMEM((NSTRUCT // NW,), jnp.float32),
        pltpu.SemaphoreType.DMA,
    ],
)
def _reduce_kernel(parts_hbm, out_hbm, stripe_v, res_v, sem):
    wid = lax.axis_index("s") * NC + lax.axis_index("c")
    cols = NSTRUCT // NW
    base = wid * cols
    descs = []
    for ch in range(cols // L):
        for r in range(NW):
            descs.append(pltpu.async_copy(
                parts_hbm.at[pl.ds(r * NSTRUCT + base + ch * L, L)],
                stripe_v.at[ch * NW + r], sem))
    for d in descs:
        d.wait()
    for ch in range(cols // L):
        tot = stripe_v[ch * NW]
        for r in range(1, NW):
            tot = tot + stripe_v[ch * NW + r]
        res_v[pl.ds(ch * L, L)] = tot
    pltpu.sync_copy(res_v, out_hbm.at[pl.ds(base, cols)])


@jax.jit
def kernel(local_energies, Z, batch, scale, shift):
    z32 = Z.astype(jnp.int32)
    b32 = batch.astype(jnp.int32)
    s_hi = jax.lax.bitcast_convert_type(
        scale.astype(jnp.bfloat16), jnp.uint16).astype(jnp.uint32) << 16
    sh_lo = jax.lax.bitcast_convert_type(
        shift.astype(jnp.bfloat16), jnp.uint16).astype(jnp.uint32)
    packed = (s_hi | sh_lo).astype(jnp.int32)
    tbl = jnp.zeros((NSPEC_PAD, L), jnp.int32)
    tbl = tbl.at[: scale.shape[0], :].set(packed[:, None])
    parts = _partials_kernel(local_energies, z32, b32, tbl.reshape(-1))
    return _reduce_kernel(parts)


# speedup vs baseline: 1.1541x; 1.0080x over previous
import functools

import jax
import jax.numpy as jnp
from jax import lax
from jax.experimental import pallas as pl
from jax.experimental.pallas import tpu as pltpu
from jax.experimental.pallas import tpu_sc as plsc

NATOMS = 6400000
NSTRUCT = 1024
NSPEC_PAD = 128

NC = 2
NS = 16
L = 16
NW = NC * NS

ATOMS_PER_W = NATOMS // NW
CHUNK = 10000
NCHUNK = ATOMS_PER_W // CHUNK
VREGS = CHUNK // L
ROWSTRIDE = NSTRUCT + L

_mesh = plsc.VectorSubcoreMesh(
    core_axis_name="c", subcore_axis_name="s", num_cores=NC, num_subcores=NS
)
_params = pltpu.CompilerParams(needs_layout_passes=False)


@functools.partial(
    pl.kernel,
    out_type=jax.ShapeDtypeStruct((NW * NSTRUCT,), jnp.float32),
    mesh=_mesh,
    compiler_params=_params,
    scratch_types=[
        pltpu.VMEM((NSPEC_PAD * L,), jnp.int32),
        pltpu.VMEM((CHUNK,), jnp.float32),
        pltpu.VMEM((CHUNK,), jnp.int32),
        pltpu.VMEM((CHUNK,), jnp.int32),
        pltpu.VMEM((CHUNK,), jnp.float32),
        pltpu.VMEM((CHUNK,), jnp.int32),
        pltpu.VMEM((CHUNK,), jnp.int32),
        pltpu.VMEM((2 * L * ROWSTRIDE,), jnp.float32),
        pltpu.VMEM((NSTRUCT,), jnp.float32),
        pltpu.SemaphoreType.DMA,
        pltpu.SemaphoreType.DMA,
    ],
)
def _partials_kernel(le_hbm, z_hbm, b_hbm, tbl_hbm, out_hbm,
                     tbl_v, le_a, z_a, b_a, le_b, z_b, b_b,
                     acc_v, fold_v, sem_a, sem_b):
    wid = lax.axis_index("s") * NC + lax.axis_index("c")

    lane = lax.iota(jnp.int32, L)
    lane_rot = lane * (ROWSTRIDE + 1)
    w_base = wid * ATOMS_PER_W

    def start(ci, le_v, z_v, b_v, sem):
        base = pl.multiple_of(w_base + ci * CHUNK, CHUNK)
        pltpu.async_copy(le_hbm.at[pl.ds(base, CHUNK)], le_v, sem)
        pltpu.async_copy(z_hbm.at[pl.ds(base, CHUNK)], z_v, sem)
        pltpu.async_copy(b_hbm.at[pl.ds(base, CHUNK)], b_v, sem)

    def wait(le_v, z_v, b_v, sem):
        pltpu.make_async_copy(le_hbm.at[pl.ds(0, CHUNK)], le_v, sem).wait()
        pltpu.make_async_copy(z_hbm.at[pl.ds(0, CHUNK)], z_v, sem).wait()
        pltpu.make_async_copy(b_hbm.at[pl.ds(0, CHUNK)], b_v, sem).wait()

    def compute(le_v, z_v, b_v):
        @plsc.parallel_loop(0, VREGS, unroll=8)
        def _(i):
            sl = pl.ds(pl.multiple_of(i * L, L), L)
            zi = (z_v[sl] << 4) + lane
            bi = b_v[sl] + lane_rot + (i & 1) * (L * ROWSTRIDE)
            pv = plsc.load_gather(tbl_v, [zi])
            s16 = plsc.bitcast(pv & jnp.int32(-65536), jnp.float32)
            sh16 = plsc.bitcast(pv << 16, jnp.float32)
            e16 = le_v[sl] * s16 + sh16
            plsc.addupdate_scatter(acc_v, [bi], e16)

    start(0, le_a, z_a, b_a, sem_a)
    tbl_d = pltpu.async_copy(tbl_hbm, tbl_v, sem_b)

    zero16 = jnp.zeros((L,), jnp.float32)

    @plsc.parallel_loop(0, 2 * L * ROWSTRIDE // L, unroll=8)
    def _(i):
        acc_v[pl.ds(pl.multiple_of(i * L, L), L)] = zero16

    tbl_d.wait()

    def pair_body(k, _):
        wait(le_a, z_a, b_a, sem_a)
        start(2 * k + 1, le_b, z_b, b_b, sem_b)
        compute(le_a, z_a, b_a)
        wait(le_b, z_b, b_b, sem_b)
        start(jnp.minimum(2 * k + 2, NCHUNK - 1), le_a, z_a, b_a, sem_a)
        compute(le_b, z_b, b_b)
        return 0

    lax.fori_loop(0, NCHUNK // 2, pair_body, 0)
    wait(le_a, z_a, b_a, sem_a)

    @plsc.parallel_loop(0, NSTRUCT // L, unroll=4)
    def _(k):
        col = k * L
        tot = acc_v[pl.ds(col, L)]
        tot = tot + acc_v[pl.ds(col + L * ROWSTRIDE, L)]
        for r in range(1, L):
            tot = tot + acc_v[pl.ds(col + r * (ROWSTRIDE + 1), L)]
            tot = tot + acc_v[pl.ds(col + r * (ROWSTRIDE + 1) + L * ROWSTRIDE, L)]
        fold_v[pl.ds(pl.multiple_of(col, L), L)] = tot

    pltpu.sync_copy(fold_v, out_hbm.at[pl.ds(wid * NSTRUCT, NSTRUCT)])


@functools.partial(
    pl.kernel,
    out_type=jax.ShapeDtypeStruct((NSTRUCT,), jnp.float32),
    mesh=_mesh,
    compiler_params=_params,
    scratch_types=[
        pltpu.VMEM((2 * NW, L), jnp.float32),
        pltpu.VMEM((NSTRUCT // NW,), jnp.float32),
        pltpu.SemaphoreType.DMA,
    ],
)
def _reduce_kernel(parts_hbm, out_hbm, stripe_v, res_v, sem):
    wid = lax.axis_index("s") * NC + lax.axis_index("c")
    cols = NSTRUCT // NW
    base = wid * cols
    descs = []
    for ch in range(cols // L):
        for r in range(NW):
            descs.append(pltpu.async_copy(
                parts_hbm.at[pl.ds(r * NSTRUCT + base + ch * L, L)],
                stripe_v.at[ch * NW + r], sem))
    for d in descs:
        d.wait()
    for ch in range(cols // L):
        tot = stripe_v[ch * NW]
        for r in range(1, NW):
            tot = tot + stripe_v[ch * NW + r]
        res_v[pl.ds(ch * L, L)] = tot
    pltpu.sync_copy(res_v, out_hbm.at[pl.ds(base, cols)])


@jax.jit
def kernel(local_energies, Z, batch, scale, shift):
    z32 = Z.astype(jnp.int32)
    b32 = batch.astype(jnp.int32)
    s_hi = jax.lax.bitcast_convert_type(
        scale.astype(jnp.bfloat16), jnp.uint16).astype(jnp.uint32) << 16
    sh_lo = jax.lax.bitcast_convert_type(
        shift.astype(jnp.bfloat16), jnp.uint16).astype(jnp.uint32)
    packed = (s_hi | sh_lo).astype(jnp.int32)
    tbl = jnp.zeros((NSPEC_PAD, L), jnp.int32)
    tbl = tbl.at[: scale.shape[0], :].set(packed[:, None])
    parts = _partials_kernel(local_energies, z32, b32, tbl.reshape(-1))
    return _reduce_kernel(parts)
